# baseline (speedup 1.0000x reference)
import functools
import math

import jax
import jax.numpy as jnp
from jax import lax
from jax.experimental import pallas as pl
from jax.experimental.pallas import tpu as pltpu
from jax.experimental.pallas import tpu_sc as plsc

D = 64
B_TOT = 4096 * 200
NC, NS = 2, 16
NW = NC * NS
B_PER_W = B_TOT // NW
GROUP = 128
GROUPS_PER_W = B_PER_W // GROUP
CHUNK_GROUPS = 4
CHUNK = GROUP * CHUNK_GROUPS
NCHUNKS = B_PER_W // CHUNK
SCALE = math.sqrt(D)
LANES = 16

_mesh = plsc.VectorSubcoreMesh(core_axis_name="c", subcore_axis_name="s")


@functools.partial(
    pl.kernel,
    mesh=_mesh,
    out_type=jax.ShapeDtypeStruct((B_TOT, D), jnp.float32),
    compiler_params=pltpu.CompilerParams(use_tc_tiling_on_sc=False),
    scratch_types=[
        pltpu.VMEM((GROUPS_PER_W, GROUP), jnp.int32),
        pltpu.VMEM((CHUNK, D), jnp.float32),
        pltpu.SemaphoreType.DMA,
    ],
)
def _emb(x_hbm, w_hbm, out_hbm, idx_v, rows_v, sem):
    wid = lax.axis_index("s") * NC + lax.axis_index("c")
    row0 = wid * B_PER_W
    pltpu.sync_copy(x_hbm.at[pl.ds(wid * GROUPS_PER_W, GROUPS_PER_W)], idx_v)

    def chunk_body(g, carry):
        for j in range(CHUNK_GROUPS):
            pltpu.async_copy(
                w_hbm.at[idx_v.at[g * CHUNK_GROUPS + j]],
                rows_v.at[pl.ds(j * GROUP, GROUP)],
                sem,
            )
        pltpu.make_async_copy(w_hbm.at[pl.ds(0, CHUNK)], rows_v, sem).wait()

        def scale_row(r, c):
            for q in range(D // LANES):
                sl = pl.ds(q * LANES, LANES)
                rows_v[r, sl] = rows_v[r, sl] * SCALE
            return c

        lax.fori_loop(0, CHUNK, scale_row, 0)
        pltpu.sync_copy(rows_v, out_hbm.at[pl.ds(row0 + g * CHUNK, CHUNK)])
        return carry

    lax.fori_loop(0, NCHUNKS, chunk_body, 0)


def kernel(x, weight):
    x2d = x.reshape(B_TOT // GROUP, GROUP)
    out = _emb(x2d, weight)
    return out.reshape(x.shape[0], x.shape[1], D)


# speedup vs baseline: 1.1159x; 1.1159x over previous
import functools
import math

import jax
import jax.numpy as jnp
from jax import lax
from jax.experimental import pallas as pl
from jax.experimental.pallas import tpu as pltpu
from jax.experimental.pallas import tpu_sc as plsc

D = 64
B_TOT = 4096 * 200
NC, NS = 2, 16
NW = NC * NS
B_PER_W = B_TOT // NW
GROUP = 128
GROUPS_PER_W = B_PER_W // GROUP
CHUNK_GROUPS = 4
CHUNK = GROUP * CHUNK_GROUPS
NCHUNKS = B_PER_W // CHUNK
NBUF = 2
SCALE = math.sqrt(D)
LANES = 16

_mesh = plsc.VectorSubcoreMesh(core_axis_name="c", subcore_axis_name="s")


@functools.partial(
    pl.kernel,
    mesh=_mesh,
    out_type=jax.ShapeDtypeStruct((B_TOT, D), jnp.float32),
    compiler_params=pltpu.CompilerParams(use_tc_tiling_on_sc=False),
    scratch_types=[
        pltpu.VMEM((GROUPS_PER_W, GROUP), jnp.int32),
        pltpu.VMEM((CHUNK, D), jnp.float32),
        pltpu.VMEM((CHUNK, D), jnp.float32),
        pltpu.SemaphoreType.DMA,
        pltpu.SemaphoreType.DMA,
        pltpu.SemaphoreType.DMA,
        pltpu.SemaphoreType.DMA,
    ],
)
def _emb(x_hbm, w_hbm, out_hbm, idx_v, rows0, rows1, gs0, gs1, os0, os1):
    wid = lax.axis_index("s") * NC + lax.axis_index("c")
    row0 = wid * B_PER_W
    rows = [rows0, rows1]
    gsem = [gs0, gs1]
    osem = [os0, os1]

    pltpu.sync_copy(x_hbm.at[pl.ds(wid * GROUPS_PER_W, GROUPS_PER_W)], idx_v)

    def start_gather(g, b):
        for j in range(CHUNK_GROUPS):
            pltpu.async_copy(
                w_hbm.at[idx_v.at[g * CHUNK_GROUPS + j]],
                rows[b].at[pl.ds(j * GROUP, GROUP)],
                gsem[b],
            )

    def wait_gather(b):
        pltpu.make_async_copy(w_hbm.at[pl.ds(0, CHUNK)], rows[b], gsem[b]).wait()

    def scale(b):
        @plsc.parallel_loop(0, CHUNK, 1, unroll=4)
        def _(r):
            for q in range(D // LANES):
                sl = pl.ds(q * LANES, LANES)
                rows[b][r, sl] = rows[b][r, sl] * SCALE

    def start_scatter(g, b):
        pltpu.async_copy(rows[b], out_hbm.at[pl.ds(row0 + g * CHUNK, CHUNK)],
                         osem[b])

    def wait_scatter(b):
        pltpu.make_async_copy(rows[b], out_hbm.at[pl.ds(row0, CHUNK)],
                              osem[b]).wait()

    for b in range(NBUF):
        start_gather(b, b)

    def pair_body(p, carry):
        for b in range(NBUF):
            g = p * NBUF + b
            wait_gather(b)
            scale(b)
            start_scatter(g, b)
            wait_scatter(b)
            start_gather(g + NBUF, b)
        return carry

    lax.fori_loop(0, NCHUNKS // NBUF - 1, pair_body, 0)

    for b in range(NBUF):
        g = NCHUNKS - NBUF + b
        wait_gather(b)
        scale(b)
        start_scatter(g, b)
        wait_scatter(b)


def kernel(x, weight):
    x2d = x.reshape(B_TOT // GROUP, GROUP)
    out = _emb(x2d, weight)
    return out.reshape(x.shape[0], x.shape[1], D)


# speedup vs baseline: 1.1191x; 1.0029x over previous
import functools
import math

import jax
import jax.numpy as jnp
from jax import lax
from jax.experimental import pallas as pl
from jax.experimental.pallas import tpu as pltpu
from jax.experimental.pallas import tpu_sc as plsc

D = 64
BATCH = 4096
SEQ = 200
NC, NS = 2, 16
NW = NC * NS
ROWS_PER_W = BATCH // NW
SPLIT = 128
REM = SEQ - SPLIT
R = 2
NCHUNKS = ROWS_PER_W // R
NBUF = 2
SCALE = math.sqrt(D)
LANES = 16

_mesh = plsc.VectorSubcoreMesh(core_axis_name="c", subcore_axis_name="s")


@functools.partial(
    pl.kernel,
    mesh=_mesh,
    out_type=jax.ShapeDtypeStruct((BATCH, SEQ, D), jnp.float32),
    compiler_params=pltpu.CompilerParams(use_tc_tiling_on_sc=False),
    scratch_types=[
        pltpu.VMEM((ROWS_PER_W, SEQ), jnp.int32),
        pltpu.VMEM((R, SEQ, D), jnp.float32),
        pltpu.VMEM((R, SEQ, D), jnp.float32),
        pltpu.SemaphoreType.DMA,
        pltpu.SemaphoreType.DMA,
        pltpu.SemaphoreType.DMA,
        pltpu.SemaphoreType.DMA,
    ],
)
def _emb(x_hbm, w_hbm, out_hbm, idx_v, rows0, rows1, gs0, gs1, os0, os1):
    wid = lax.axis_index("s") * NC + lax.axis_index("c")
    xr0 = wid * ROWS_PER_W
    rows = [rows0, rows1]
    gsem = [gs0, gs1]
    osem = [os0, os1]

    pltpu.sync_copy(x_hbm.at[pl.ds(xr0, ROWS_PER_W)], idx_v)

    def start_gather(g, b):
        for rr in range(R):
            row = g * R + rr
            pltpu.async_copy(
                w_hbm.at[idx_v.at[row, pl.ds(0, SPLIT)]],
                rows[b].at[rr, pl.ds(0, SPLIT)],
                gsem[b],
            )
            pltpu.async_copy(
                w_hbm.at[idx_v.at[row, pl.ds(SPLIT, REM)]],
                rows[b].at[rr, pl.ds(SPLIT, REM)],
                gsem[b],
            )

    def wait_gather(b):
        pltpu.make_async_copy(out_hbm.at[pl.ds(0, R)], rows[b], gsem[b]).wait()

    def scale(b):
        @plsc.parallel_loop(0, SEQ, 1, unroll=4)
        def _(c):
            for rr in range(R):
                for q in range(D // LANES):
                    sl = pl.ds(q * LANES, LANES)
                    rows[b][rr, c, sl] = rows[b][rr, c, sl] * SCALE

    def start_scatter(g, b):
        pltpu.async_copy(rows[b], out_hbm.at[pl.ds(xr0 + g * R, R)], osem[b])

    def wait_scatter(b):
        pltpu.make_async_copy(rows[b], out_hbm.at[pl.ds(xr0, R)],
                              osem[b]).wait()

    for b in range(NBUF):
        start_gather(b, b)

    def pair_body(p, carry):
        for b in range(NBUF):
            g = p * NBUF + b
            wait_gather(b)
            scale(b)
            start_scatter(g, b)
            wait_scatter(b)
            start_gather(g + NBUF, b)
        return carry

    lax.fori_loop(0, NCHUNKS // NBUF - 1, pair_body, 0)

    for b in range(NBUF):
        g = NCHUNKS - NBUF + b
        wait_gather(b)
        scale(b)
        start_scatter(g, b)
        wait_scatter(b)


def kernel(x, weight):
    return _emb(x, weight)


# speedup vs baseline: 1.3656x; 1.2203x over previous
import functools
import math

import jax
import jax.numpy as jnp
from jax import lax
from jax.experimental import pallas as pl
from jax.experimental.pallas import tpu as pltpu
from jax.experimental.pallas import tpu_sc as plsc

D = 64
DP = 128
BATCH = 4096
SEQ = 200
VOCAB = 1000000
NC, NS = 2, 16
NW = NC * NS
ROWS_PER_W = BATCH // NW
SPLIT = 128
REM = SEQ - SPLIT
NBUF = 2
SCALE = math.sqrt(D)
LANES = 16

_mesh = plsc.VectorSubcoreMesh(core_axis_name="c", subcore_axis_name="s")


@functools.partial(
    pl.kernel,
    mesh=_mesh,
    out_type=jax.ShapeDtypeStruct((BATCH, SEQ, DP), jnp.float32),
    compiler_params=pltpu.CompilerParams(use_tc_tiling_on_sc=True),
    scratch_types=[
        pltpu.VMEM((ROWS_PER_W, SEQ), jnp.int32),
        pltpu.VMEM((SEQ, DP), jnp.float32),
        pltpu.VMEM((SEQ, DP), jnp.float32),
        pltpu.SemaphoreType.DMA,
        pltpu.SemaphoreType.DMA,
        pltpu.SemaphoreType.DMA,
        pltpu.SemaphoreType.DMA,
    ],
)
def _emb(x_hbm, w_hbm, out_hbm, idx_v, rows0, rows1, gs0, gs1, os0, os1):
    wid = lax.axis_index("s") * NC + lax.axis_index("c")
    xr0 = wid * ROWS_PER_W
    rows = [rows0, rows1]
    gsem = [gs0, gs1]
    osem = [os0, os1]

    pltpu.sync_copy(x_hbm.at[pl.ds(xr0, ROWS_PER_W)], idx_v)

    def start_gather(g, b):
        pltpu.async_copy(
            w_hbm.at[idx_v.at[g, pl.ds(0, SPLIT)]],
            rows[b].at[pl.ds(0, SPLIT)],
            gsem[b],
        )
        pltpu.async_copy(
            w_hbm.at[idx_v.at[g, pl.ds(SPLIT, REM)]],
            rows[b].at[pl.ds(SPLIT, REM)],
            gsem[b],
        )

    def wait_gather(b):
        pltpu.make_async_copy(w_hbm.at[pl.ds(0, SEQ)], rows[b], gsem[b]).wait()

    def scale(b):
        @plsc.parallel_loop(0, SEQ, 1, unroll=4)
        def _(c):
            for q in range(D // LANES):
                sl = pl.ds(q * LANES, LANES)
                rows[b][c, sl] = rows[b][c, sl] * SCALE

    def start_scatter(g, b):
        pltpu.async_copy(rows[b], out_hbm.at[xr0 + g], osem[b])

    def wait_scatter(b):
        pltpu.make_async_copy(rows[b], out_hbm.at[xr0], osem[b]).wait()

    for b in range(NBUF):
        start_gather(b, b)

    def pair_body(p, carry):
        for b in range(NBUF):
            g = p * NBUF + b
            wait_gather(b)
            scale(b)
            start_scatter(g, b)
            wait_scatter(b)
            start_gather(g + NBUF, b)
        return carry

    lax.fori_loop(0, ROWS_PER_W // NBUF - 1, pair_body, 0)

    for b in range(NBUF):
        g = ROWS_PER_W - NBUF + b
        wait_gather(b)
        scale(b)
        start_scatter(g, b)
        wait_scatter(b)


def kernel(x, weight):
    w128 = jnp.pad(weight, ((0, 0), (0, DP - D)))
    out = _emb(x, w128)
    return out[:, :, :D]
